# baseline (speedup 1.0000x reference)
import functools

import jax
import jax.numpy as jnp
from jax import lax
from jax.experimental import pallas as pl
from jax.experimental.pallas import tpu as pltpu
from jax.experimental.pallas import tpu_sc as plsc

N_NODES = 10000
N_PAD = 10240
D_FEAT = 128
N_EDGES = 320000
NUM_CORES = 2
NUM_SUBCORES = 16
NUM_WORKERS = NUM_CORES * NUM_SUBCORES
EDGES_PER_WORKER = N_EDGES // NUM_WORKERS
EPW_PAD = 10240
CHUNK = 40
NUM_CHUNKS = EPW_PAD // CHUNK
NBUF = 4
AHEAD = 2
GROUPS = EDGES_PER_WORKER // 16
ROWS_PER_TILE = N_PAD // NUM_SUBCORES


def _make_mesh():
    return plsc.VectorSubcoreMesh(core_axis_name="c", subcore_axis_name="s")


def _make_agg():
    @functools.partial(
        pl.kernel,
        mesh=_make_mesh(),
        compiler_params=pltpu.CompilerParams(use_tc_tiling_on_sc=False),
        out_type=jax.ShapeDtypeStruct((NUM_CORES, N_PAD, D_FEAT), jnp.float32),
        scratch_types=[
            pltpu.VMEM((NUM_CHUNKS, CHUNK), jnp.int32),
            pltpu.VMEM((NUM_CHUNKS, CHUNK), jnp.int32),
            [pltpu.VMEM((CHUNK, D_FEAT), jnp.float32) for _ in range(NBUF)],
            pltpu.VMEM_SHARED((N_PAD, D_FEAT), jnp.float32),
            [pltpu.SemaphoreType.DMA for _ in range(NBUF)],
            [pltpu.SemaphoreType.DMA for _ in range(NBUF)],
        ],
    )
    def agg(
        x_hbm, src_hbm, dst_hbm, zeros_hbm, out_hbm,
        src_v, dst_v, rows, acc_sh, gsem, ssem,
    ):
        cid = lax.axis_index("c")
        sid = lax.axis_index("s")
        wid = cid * NUM_SUBCORES + sid
        row0 = sid * ROWS_PER_TILE
        pltpu.sync_copy(
            zeros_hbm.at[pl.ds(row0, ROWS_PER_TILE)],
            acc_sh.at[pl.ds(row0, ROWS_PER_TILE)],
        )
        pltpu.sync_copy(src_hbm.at[wid], src_v)
        pltpu.sync_copy(dst_hbm.at[wid], dst_v)
        plsc.subcore_barrier()

        def gather(c, b):
            return pltpu.make_async_copy(x_hbm.at[src_v.at[c]], rows[b], gsem[b])

        def scatter(c, b):
            return pltpu.make_async_copy(rows[b], acc_sh.at[dst_v.at[c]], ssem[b])

        gather(0, 0).start()
        gather(1, 1).start()

        def body(o, carry):
            for j in range(NBUF):
                c = NBUF * o + j
                b = j
                b2 = (j + AHEAD) % NBUF
                cn = c + AHEAD

                @pl.when(jnp.logical_and(cn < NUM_CHUNKS, c >= AHEAD))
                def _():
                    scatter(c - AHEAD, b2).wait()

                @pl.when(cn < NUM_CHUNKS)
                def _():
                    gather(cn, b2).start()

                gather(c, b).wait()
                scatter(c, b).start(add=True)
            return carry

        lax.fori_loop(0, NUM_CHUNKS // NBUF, body, 0)
        for k in range(NBUF):
            c = NUM_CHUNKS - NBUF + k
            scatter(c, c % NBUF).wait()
        plsc.subcore_barrier()
        pltpu.sync_copy(
            acc_sh.at[pl.ds(row0, ROWS_PER_TILE)],
            out_hbm.at[cid, pl.ds(row0, ROWS_PER_TILE)],
        )

    return agg


def _make_deg():
    @functools.partial(
        pl.kernel,
        mesh=_make_mesh(),
        compiler_params=pltpu.CompilerParams(needs_layout_passes=False),
        out_type=jax.ShapeDtypeStruct((NUM_WORKERS, N_PAD), jnp.float32),
        scratch_types=[
            pltpu.VMEM((GROUPS, 16), jnp.int32),
            pltpu.VMEM((N_PAD,), jnp.float32),
        ],
    )
    def deg(dst_hbm, zeros_hbm, out_hbm, dst_v, hist_v):
        cid = lax.axis_index("c")
        sid = lax.axis_index("s")
        wid = cid * NUM_SUBCORES + sid
        pltpu.sync_copy(zeros_hbm, hist_v)
        pltpu.sync_copy(dst_hbm.at[wid], dst_v)
        ones = jnp.full((16,), 1.0, jnp.float32)

        def body(g, carry):
            idx = dst_v[g, :]
            plsc.addupdate_scatter(hist_v, [idx], ones)
            return carry

        lax.fori_loop(0, GROUPS, body, 0)
        pltpu.sync_copy(hist_v, out_hbm.at[wid])

    return deg


_BLK = 1024


def _make_linear(d_out, relu):
    def body(m0_ref, m1_ref, d_ref, w_ref, b_ref, o_ref):
        m = m0_ref[...] + m1_ref[...]
        deg = jnp.sum(d_ref[...], axis=0)[:, None]
        deg = jnp.maximum(deg, 1.0)
        h = jnp.dot(m / deg, w_ref[...], preferred_element_type=jnp.float32)
        h = h + b_ref[...]
        if relu:
            h = jnp.maximum(h, 0.0)
        o_ref[...] = h

    return pl.pallas_call(
        body,
        grid=(N_PAD // _BLK,),
        in_specs=[
            pl.BlockSpec((_BLK, D_FEAT), lambda i: (i, 0)),
            pl.BlockSpec((_BLK, D_FEAT), lambda i: (i, 0)),
            pl.BlockSpec((NUM_WORKERS, _BLK), lambda i: (0, i)),
            pl.BlockSpec((D_FEAT, d_out), lambda i: (0, 0)),
            pl.BlockSpec((1, d_out), lambda i: (0, 0)),
        ],
        out_specs=pl.BlockSpec((_BLK, d_out), lambda i: (i, 0)),
        out_shape=jax.ShapeDtypeStruct((N_PAD, d_out), jnp.float32),
    )


def kernel(x, edge_index, W0, b0, W1, b1):
    pad = EPW_PAD - EDGES_PER_WORKER
    src = jnp.pad(
        edge_index[0].reshape(NUM_WORKERS, EDGES_PER_WORKER), ((0, 0), (0, pad))
    ).reshape(NUM_WORKERS, NUM_CHUNKS, CHUNK)
    dst = jnp.pad(
        edge_index[1].reshape(NUM_WORKERS, EDGES_PER_WORKER),
        ((0, 0), (0, pad)),
        constant_values=N_NODES,
    ).reshape(NUM_WORKERS, NUM_CHUNKS, CHUNK)
    dst_g = edge_index[1].reshape(NUM_WORKERS, GROUPS, 16)

    zeros_nd = jnp.zeros((N_PAD, D_FEAT), jnp.float32)
    zeros_n = jnp.zeros((N_PAD,), jnp.float32)

    agg = _make_agg()
    degp = _make_deg()(dst_g, zeros_n)
    mp = agg(x, src, dst, zeros_nd)
    h = _make_linear(D_FEAT, True)(mp[0], mp[1], degp, W0, b0.reshape(1, D_FEAT))
    mp2 = agg(h, src, dst, zeros_nd)
    n_classes = W1.shape[1]
    out = _make_linear(n_classes, False)(
        mp2[0], mp2[1], degp, W1, b1.reshape(1, n_classes)
    )
    return out[:N_NODES]


# speedup vs baseline: 1.0002x; 1.0002x over previous
import functools

import jax
import jax.numpy as jnp
from jax import lax
from jax.experimental import pallas as pl
from jax.experimental.pallas import tpu as pltpu
from jax.experimental.pallas import tpu_sc as plsc

N_NODES = 10000
N_PAD = 10240
D_FEAT = 128
N_EDGES = 320000
NUM_CORES = 2
NUM_SUBCORES = 16
NUM_WORKERS = NUM_CORES * NUM_SUBCORES
EDGES_PER_WORKER = N_EDGES // NUM_WORKERS
EPW_PAD = 10240
CHUNK = 40
NUM_CHUNKS = EPW_PAD // CHUNK
NBUF = 4
AHEAD = 2
GROUPS = EDGES_PER_WORKER // 16
ROWS_PER_TILE = N_PAD // NUM_SUBCORES


def _make_mesh():
    return plsc.VectorSubcoreMesh(core_axis_name="c", subcore_axis_name="s")


def _make_agg():
    @functools.partial(
        pl.kernel,
        mesh=_make_mesh(),
        compiler_params=pltpu.CompilerParams(use_tc_tiling_on_sc=False),
        out_type=jax.ShapeDtypeStruct((NUM_CORES, N_PAD, D_FEAT), jnp.float32),
        scratch_types=[
            pltpu.VMEM((NUM_CHUNKS, CHUNK), jnp.int32),
            pltpu.VMEM((NUM_CHUNKS, CHUNK), jnp.int32),
            [pltpu.VMEM((CHUNK, D_FEAT), jnp.float32) for _ in range(NBUF)],
            pltpu.VMEM_SHARED((N_PAD, D_FEAT), jnp.float32),
            [pltpu.SemaphoreType.DMA for _ in range(NBUF)],
            [pltpu.SemaphoreType.DMA for _ in range(NBUF)],
        ],
    )
    def agg(
        x_hbm, src_hbm, dst_hbm, zeros_hbm, out_hbm,
        src_v, dst_v, rows, acc_sh, gsem, ssem,
    ):
        cid = lax.axis_index("c")
        sid = lax.axis_index("s")
        wid = cid * NUM_SUBCORES + sid
        row0 = sid * ROWS_PER_TILE
        pltpu.sync_copy(
            zeros_hbm.at[pl.ds(row0, ROWS_PER_TILE)],
            acc_sh.at[pl.ds(row0, ROWS_PER_TILE)],
        )
        pltpu.sync_copy(src_hbm.at[wid], src_v)
        pltpu.sync_copy(dst_hbm.at[wid], dst_v)
        plsc.subcore_barrier()

        def gather(c, b):
            return pltpu.make_async_copy(x_hbm.at[src_v.at[c]], rows[b], gsem[b])

        def scatter(c, b):
            return pltpu.make_async_copy(rows[b], acc_sh.at[dst_v.at[c]], ssem[b])

        def slot(c, j, first, last):
            b = j
            b2 = (j + AHEAD) % NBUF
            if not first and not last:
                scatter(c - AHEAD, b2).wait()
            if not last:
                gather(c + AHEAD, b2).start()
            gather(c, b).wait()
            scatter(c, b).start(add=True)

        gather(0, 0).start()
        gather(1, 1).start()
        for j in range(NBUF):
            slot(j, j, first=(j < AHEAD), last=False)

        def body(o, carry):
            c0 = NBUF * o
            for j in range(NBUF):
                slot(c0 + j, j, first=False, last=False)
            return carry

        lax.fori_loop(1, NUM_CHUNKS // NBUF - 1, body, 0)
        for j in range(NBUF):
            c = NUM_CHUNKS - NBUF + j
            scatter(c - AHEAD, (j + AHEAD) % NBUF).wait()
            if c + AHEAD < NUM_CHUNKS:
                gather(c + AHEAD, (j + AHEAD) % NBUF).start()
            gather(c, j).wait()
            scatter(c, j).start(add=True)
        for k in range(AHEAD):
            c = NUM_CHUNKS - AHEAD + k
            scatter(c, c % NBUF).wait()
        plsc.subcore_barrier()
        pltpu.sync_copy(
            acc_sh.at[pl.ds(row0, ROWS_PER_TILE)],
            out_hbm.at[cid, pl.ds(row0, ROWS_PER_TILE)],
        )

    return agg


def _make_deg():
    @functools.partial(
        pl.kernel,
        mesh=_make_mesh(),
        compiler_params=pltpu.CompilerParams(needs_layout_passes=False),
        out_type=jax.ShapeDtypeStruct((NUM_WORKERS, N_PAD), jnp.float32),
        scratch_types=[
            pltpu.VMEM((GROUPS, 16), jnp.int32),
            pltpu.VMEM((N_PAD,), jnp.float32),
        ],
    )
    def deg(dst_hbm, zeros_hbm, out_hbm, dst_v, hist_v):
        cid = lax.axis_index("c")
        sid = lax.axis_index("s")
        wid = cid * NUM_SUBCORES + sid
        pltpu.sync_copy(zeros_hbm, hist_v)
        pltpu.sync_copy(dst_hbm.at[wid], dst_v)
        ones = jnp.full((16,), 1.0, jnp.float32)

        def body(g, carry):
            idx = dst_v[g, :]
            plsc.addupdate_scatter(hist_v, [idx], ones)
            return carry

        lax.fori_loop(0, GROUPS, body, 0)
        pltpu.sync_copy(hist_v, out_hbm.at[wid])

    return deg


_BLK = 1024


def _make_linear(d_out, relu):
    def body(m0_ref, m1_ref, d_ref, w_ref, b_ref, o_ref):
        m = m0_ref[...] + m1_ref[...]
        deg = jnp.sum(d_ref[...], axis=0)[:, None]
        deg = jnp.maximum(deg, 1.0)
        h = jnp.dot(m / deg, w_ref[...], preferred_element_type=jnp.float32)
        h = h + b_ref[...]
        if relu:
            h = jnp.maximum(h, 0.0)
        o_ref[...] = h

    return pl.pallas_call(
        body,
        grid=(N_PAD // _BLK,),
        in_specs=[
            pl.BlockSpec((_BLK, D_FEAT), lambda i: (i, 0)),
            pl.BlockSpec((_BLK, D_FEAT), lambda i: (i, 0)),
            pl.BlockSpec((NUM_WORKERS, _BLK), lambda i: (0, i)),
            pl.BlockSpec((D_FEAT, d_out), lambda i: (0, 0)),
            pl.BlockSpec((1, d_out), lambda i: (0, 0)),
        ],
        out_specs=pl.BlockSpec((_BLK, d_out), lambda i: (i, 0)),
        out_shape=jax.ShapeDtypeStruct((N_PAD, d_out), jnp.float32),
    )


def kernel(x, edge_index, W0, b0, W1, b1):
    pad = EPW_PAD - EDGES_PER_WORKER
    src = jnp.pad(
        edge_index[0].reshape(NUM_WORKERS, EDGES_PER_WORKER), ((0, 0), (0, pad))
    ).reshape(NUM_WORKERS, NUM_CHUNKS, CHUNK)
    dst = jnp.pad(
        edge_index[1].reshape(NUM_WORKERS, EDGES_PER_WORKER),
        ((0, 0), (0, pad)),
        constant_values=N_NODES,
    ).reshape(NUM_WORKERS, NUM_CHUNKS, CHUNK)
    dst_g = edge_index[1].reshape(NUM_WORKERS, GROUPS, 16)

    zeros_nd = jnp.zeros((N_PAD, D_FEAT), jnp.float32)
    zeros_n = jnp.zeros((N_PAD,), jnp.float32)

    agg = _make_agg()
    degp = _make_deg()(dst_g, zeros_n)
    mp = agg(x, src, dst, zeros_nd)
    h = _make_linear(D_FEAT, True)(mp[0], mp[1], degp, W0, b0.reshape(1, D_FEAT))
    mp2 = agg(h, src, dst, zeros_nd)
    n_classes = W1.shape[1]
    out = _make_linear(n_classes, False)(
        mp2[0], mp2[1], degp, W1, b1.reshape(1, n_classes)
    )
    return out[:N_NODES]


# speedup vs baseline: 1.7485x; 1.7482x over previous
import functools

import jax
import jax.numpy as jnp
from jax import lax
from jax.experimental import pallas as pl
from jax.experimental.pallas import tpu as pltpu
from jax.experimental.pallas import tpu_sc as plsc

N_NODES = 10000
N_PAD = 10240
D_FEAT = 128
N_EDGES = 320000
NUM_CORES = 2
NUM_SUBCORES = 16
NUM_WORKERS = NUM_CORES * NUM_SUBCORES
EDGES_PER_WORKER = N_EDGES // NUM_WORKERS
EPW_PAD = 10080
CHUNK = 96
NUM_CHUNKS = EPW_PAD // CHUNK
GROUPS = EDGES_PER_WORKER // 16
ROWS_PER_TILE = N_PAD // NUM_SUBCORES


def _make_mesh():
    return plsc.VectorSubcoreMesh(core_axis_name="c", subcore_axis_name="s")


def _make_agg():
    @functools.partial(
        pl.kernel,
        mesh=_make_mesh(),
        compiler_params=pltpu.CompilerParams(use_tc_tiling_on_sc=False),
        out_type=jax.ShapeDtypeStruct((NUM_CORES, N_PAD, D_FEAT), jnp.float32),
        scratch_types=[
            pltpu.VMEM((NUM_CHUNKS, CHUNK), jnp.int32),
            pltpu.VMEM((NUM_CHUNKS, CHUNK), jnp.int32),
            pltpu.VMEM((CHUNK, D_FEAT), jnp.float32),
            pltpu.VMEM((CHUNK, D_FEAT), jnp.float32),
            pltpu.VMEM_SHARED((N_PAD, D_FEAT), jnp.float32),
            pltpu.SemaphoreType.DMA,
            pltpu.SemaphoreType.DMA,
        ],
    )
    def agg(
        x_hbm, src_hbm, dst_hbm, zeros_hbm, out_hbm,
        src_v, dst_v, rows_a, rows_b, acc_sh, sem_a, sem_b,
    ):
        cid = lax.axis_index("c")
        sid = lax.axis_index("s")
        wid = cid * NUM_SUBCORES + sid
        row0 = sid * ROWS_PER_TILE
        pltpu.sync_copy(
            zeros_hbm.at[pl.ds(row0, ROWS_PER_TILE)],
            acc_sh.at[pl.ds(row0, ROWS_PER_TILE)],
        )
        pltpu.sync_copy(src_hbm.at[wid], src_v)
        pltpu.sync_copy(dst_hbm.at[wid], dst_v)
        plsc.subcore_barrier()

        def gather(c, rows, sem):
            return pltpu.make_async_copy(x_hbm.at[src_v.at[c]], rows, sem)

        def scatter(c, rows):
            pltpu.sync_copy(rows, acc_sh.at[dst_v.at[c]], add=True)

        gather(0, rows_a, sem_a).start()

        def body(g, carry):
            c1 = 2 * g + 1
            gather(c1, rows_b, sem_b).start()
            gather(c1 - 1, rows_a, sem_a).wait()
            scatter(c1 - 1, rows_a)
            gather(c1 + 1, rows_a, sem_a).start()
            gather(c1, rows_b, sem_b).wait()
            scatter(c1, rows_b)
            return carry

        lax.fori_loop(0, (NUM_CHUNKS - 1) // 2, body, 0)
        gather(NUM_CHUNKS - 1, rows_a, sem_a).wait()
        scatter(NUM_CHUNKS - 1, rows_a)
        plsc.subcore_barrier()
        pltpu.sync_copy(
            acc_sh.at[pl.ds(row0, ROWS_PER_TILE)],
            out_hbm.at[cid, pl.ds(row0, ROWS_PER_TILE)],
        )

    return agg


def _make_deg():
    @functools.partial(
        pl.kernel,
        mesh=_make_mesh(),
        compiler_params=pltpu.CompilerParams(needs_layout_passes=False),
        out_type=jax.ShapeDtypeStruct((NUM_WORKERS, N_PAD), jnp.float32),
        scratch_types=[
            pltpu.VMEM((GROUPS, 16), jnp.int32),
            pltpu.VMEM((N_PAD,), jnp.float32),
        ],
    )
    def deg(dst_hbm, zeros_hbm, out_hbm, dst_v, hist_v):
        cid = lax.axis_index("c")
        sid = lax.axis_index("s")
        wid = cid * NUM_SUBCORES + sid
        pltpu.sync_copy(zeros_hbm, hist_v)
        pltpu.sync_copy(dst_hbm.at[wid], dst_v)
        ones = jnp.full((16,), 1.0, jnp.float32)

        def body(g, carry):
            idx = dst_v[g, :]
            plsc.addupdate_scatter(hist_v, [idx], ones)
            return carry

        lax.fori_loop(0, GROUPS, body, 0)
        pltpu.sync_copy(hist_v, out_hbm.at[wid])

    return deg


_BLK = 1024


def _make_linear(d_out, relu):
    def body(m0_ref, m1_ref, d_ref, w_ref, b_ref, o_ref):
        m = m0_ref[...] + m1_ref[...]
        deg = jnp.sum(d_ref[...], axis=0)[:, None]
        deg = jnp.maximum(deg, 1.0)
        h = jnp.dot(m / deg, w_ref[...], preferred_element_type=jnp.float32)
        h = h + b_ref[...]
        if relu:
            h = jnp.maximum(h, 0.0)
        o_ref[...] = h

    return pl.pallas_call(
        body,
        grid=(N_PAD // _BLK,),
        in_specs=[
            pl.BlockSpec((_BLK, D_FEAT), lambda i: (i, 0)),
            pl.BlockSpec((_BLK, D_FEAT), lambda i: (i, 0)),
            pl.BlockSpec((NUM_WORKERS, _BLK), lambda i: (0, i)),
            pl.BlockSpec((D_FEAT, d_out), lambda i: (0, 0)),
            pl.BlockSpec((1, d_out), lambda i: (0, 0)),
        ],
        out_specs=pl.BlockSpec((_BLK, d_out), lambda i: (i, 0)),
        out_shape=jax.ShapeDtypeStruct((N_PAD, d_out), jnp.float32),
    )


def kernel(x, edge_index, W0, b0, W1, b1):
    pad = EPW_PAD - EDGES_PER_WORKER
    src = jnp.pad(
        edge_index[0].reshape(NUM_WORKERS, EDGES_PER_WORKER), ((0, 0), (0, pad))
    ).reshape(NUM_WORKERS, NUM_CHUNKS, CHUNK)
    dst = jnp.pad(
        edge_index[1].reshape(NUM_WORKERS, EDGES_PER_WORKER),
        ((0, 0), (0, pad)),
        constant_values=N_NODES,
    ).reshape(NUM_WORKERS, NUM_CHUNKS, CHUNK)
    dst_g = edge_index[1].reshape(NUM_WORKERS, GROUPS, 16)

    zeros_nd = jnp.zeros((N_PAD, D_FEAT), jnp.float32)
    zeros_n = jnp.zeros((N_PAD,), jnp.float32)

    agg = _make_agg()
    degp = _make_deg()(dst_g, zeros_n)
    mp = agg(x, src, dst, zeros_nd)
    h = _make_linear(D_FEAT, True)(mp[0], mp[1], degp, W0, b0.reshape(1, D_FEAT))
    mp2 = agg(h, src, dst, zeros_nd)
    n_classes = W1.shape[1]
    out = _make_linear(n_classes, False)(
        mp2[0], mp2[1], degp, W1, b1.reshape(1, n_classes)
    )
    return out[:N_NODES]


# speedup vs baseline: 2.7040x; 1.5465x over previous
import functools

import jax
import jax.numpy as jnp
from jax import lax
from jax.experimental import pallas as pl
from jax.experimental.pallas import tpu as pltpu
from jax.experimental.pallas import tpu_sc as plsc

N_NODES = 10000
N_PAD = 10240
D_FEAT = 128
N_EDGES = 320000
NUM_CORES = 2
NUM_SUBCORES = 16
NUM_WORKERS = NUM_CORES * NUM_SUBCORES
EDGES_PER_WORKER = N_EDGES // NUM_WORKERS
EPW_PAD = 10000
CHUNK = 80
NUM_CHUNKS = EPW_PAD // CHUNK
GROUPS = EDGES_PER_WORKER // 16
ROWS_PER_TILE = N_PAD // NUM_SUBCORES


def _make_mesh():
    return plsc.VectorSubcoreMesh(core_axis_name="c", subcore_axis_name="s")


def _make_agg():
    @functools.partial(
        pl.kernel,
        mesh=_make_mesh(),
        compiler_params=pltpu.CompilerParams(use_tc_tiling_on_sc=False),
        out_type=jax.ShapeDtypeStruct((NUM_CORES, N_PAD, D_FEAT), jnp.float32),
        scratch_types=[
            pltpu.VMEM((NUM_CHUNKS, CHUNK), jnp.int32),
            pltpu.VMEM((NUM_CHUNKS, CHUNK), jnp.int32),
            pltpu.VMEM((CHUNK, D_FEAT), jnp.float32),
            pltpu.VMEM((CHUNK, D_FEAT), jnp.float32),
            pltpu.VMEM_SHARED((N_PAD, D_FEAT), jnp.float32),
            pltpu.SemaphoreType.DMA,
            pltpu.SemaphoreType.DMA,
        ],
    )
    def agg(
        x_hbm, src_hbm, dst_hbm, zeros_hbm, out_hbm,
        src_v, dst_v, rows_a, rows_b, acc_sh, sem_a, sem_b,
    ):
        cid = lax.axis_index("c")
        sid = lax.axis_index("s")
        wid = cid * NUM_SUBCORES + sid
        row0 = sid * ROWS_PER_TILE
        pltpu.sync_copy(
            zeros_hbm.at[pl.ds(row0, ROWS_PER_TILE)],
            acc_sh.at[pl.ds(row0, ROWS_PER_TILE)],
        )
        pltpu.sync_copy(src_hbm.at[wid], src_v)
        pltpu.sync_copy(dst_hbm.at[wid], dst_v)
        plsc.subcore_barrier()

        def gather(c, rows, sem):
            return pltpu.make_async_copy(x_hbm.at[src_v.at[c]], rows, sem)

        def scatter(c, rows):
            pltpu.sync_copy(rows, acc_sh.at[dst_v.at[c]], add=True)

        gather(0, rows_a, sem_a).start()

        def body(g, carry):
            c1 = 2 * g + 1
            gather(c1, rows_b, sem_b).start()
            gather(c1 - 1, rows_a, sem_a).wait()
            scatter(c1 - 1, rows_a)
            gather(c1 + 1, rows_a, sem_a).start()
            gather(c1, rows_b, sem_b).wait()
            scatter(c1, rows_b)
            return carry

        lax.fori_loop(0, (NUM_CHUNKS - 1) // 2, body, 0)
        gather(NUM_CHUNKS - 1, rows_a, sem_a).wait()
        scatter(NUM_CHUNKS - 1, rows_a)
        plsc.subcore_barrier()
        pltpu.sync_copy(
            acc_sh.at[pl.ds(row0, ROWS_PER_TILE)],
            out_hbm.at[cid, pl.ds(row0, ROWS_PER_TILE)],
        )

    return agg


def _make_deg():
    @functools.partial(
        pl.kernel,
        mesh=_make_mesh(),
        compiler_params=pltpu.CompilerParams(needs_layout_passes=False),
        out_type=jax.ShapeDtypeStruct((NUM_WORKERS, N_PAD), jnp.float32),
        scratch_types=[
            pltpu.VMEM((GROUPS, 16), jnp.int32),
            pltpu.VMEM((N_PAD,), jnp.float32),
        ],
    )
    def deg(dst_hbm, zeros_hbm, out_hbm, dst_v, hist_v):
        cid = lax.axis_index("c")
        sid = lax.axis_index("s")
        wid = cid * NUM_SUBCORES + sid
        pltpu.sync_copy(zeros_hbm, hist_v)
        pltpu.sync_copy(dst_hbm.at[wid], dst_v)
        ones = jnp.full((16,), 1.0, jnp.float32)

        def body(g, carry):
            idx = dst_v[g, :]
            plsc.addupdate_scatter(hist_v, [idx], ones)
            return carry

        lax.fori_loop(0, GROUPS, body, 0)
        pltpu.sync_copy(hist_v, out_hbm.at[wid])

    return deg


_BLK = 1024


def _make_linear(d_out, relu):
    def body(m0_ref, m1_ref, d_ref, w_ref, b_ref, o_ref):
        m = m0_ref[...] + m1_ref[...]
        deg = jnp.sum(d_ref[...], axis=0)[:, None]
        deg = jnp.maximum(deg, 1.0)
        h = jnp.dot(m / deg, w_ref[...], preferred_element_type=jnp.float32)
        h = h + b_ref[...]
        if relu:
            h = jnp.maximum(h, 0.0)
        o_ref[...] = h

    return pl.pallas_call(
        body,
        grid=(N_PAD // _BLK,),
        in_specs=[
            pl.BlockSpec((_BLK, D_FEAT), lambda i: (i, 0)),
            pl.BlockSpec((_BLK, D_FEAT), lambda i: (i, 0)),
            pl.BlockSpec((NUM_WORKERS, _BLK), lambda i: (0, i)),
            pl.BlockSpec((D_FEAT, d_out), lambda i: (0, 0)),
            pl.BlockSpec((1, d_out), lambda i: (0, 0)),
        ],
        out_specs=pl.BlockSpec((_BLK, d_out), lambda i: (i, 0)),
        out_shape=jax.ShapeDtypeStruct((N_PAD, d_out), jnp.float32),
    )


def kernel(x, edge_index, W0, b0, W1, b1):
    src = edge_index[0].reshape(NUM_WORKERS, NUM_CHUNKS, CHUNK)
    dst = edge_index[1].reshape(NUM_WORKERS, NUM_CHUNKS, CHUNK)
    dst_g = edge_index[1].reshape(NUM_WORKERS, GROUPS, 16)

    zeros_nd = jnp.zeros((N_PAD, D_FEAT), jnp.float32)
    zeros_n = jnp.zeros((N_PAD,), jnp.float32)

    agg = _make_agg()
    degp = _make_deg()(dst_g, zeros_n)
    mp = agg(x, src, dst, zeros_nd)
    h = _make_linear(D_FEAT, True)(mp[0], mp[1], degp, W0, b0.reshape(1, D_FEAT))
    mp2 = agg(h, src, dst, zeros_nd)
    n_classes = W1.shape[1]
    out = _make_linear(n_classes, False)(
        mp2[0], mp2[1], degp, W1, b1.reshape(1, n_classes)
    )
    return out[:N_NODES]
